# baseline (speedup 1.0000x reference)
import functools

import jax
import jax.numpy as jnp
from jax import lax
from jax.experimental import pallas as pl
from jax.experimental.pallas import tpu as pltpu
from jax.experimental.pallas import tpu_sc as plsc

NC = 2
NS = 16
NW = NC * NS
CH = 96
GRP = 35


def _sc_agg_body(rpt, hr_hbm, row_hbm, col_hbm, out_hbm,
                 row_v, col_v, ga, gb, agg_sh, sema, semb):
  cid = lax.axis_index("c")
  sid = lax.axis_index("s")
  wid = sid * NC + cid

  zeros = jnp.zeros((16,), jnp.float32)
  zr, zc = ga.shape

  def zero_body(r, _):
    for c8 in range(zc // 16):
      ga[r, pl.ds(c8 * 16, 16)] = zeros
    return ()
  lax.fori_loop(0, zr, zero_body, ())

  myrow = sid * rpt
  for t in range(rpt // zr):
    pltpu.sync_copy(ga, agg_sh.at[pl.ds(myrow + t * zr, zr)])
  rem = rpt % zr
  if rem:
    pltpu.sync_copy(ga.at[pl.ds(0, rem)],
                    agg_sh.at[pl.ds(myrow + (rpt // zr) * zr, rem)])

  plsc.subcore_barrier()

  ngrp = row_hbm.shape[1]
  grp = row_hbm.shape[2]

  def group_body(g, _):
    pltpu.sync_copy(row_hbm.at[wid, g], row_v)
    pltpu.sync_copy(col_hbm.at[wid, g], col_v)

    pltpu.async_copy(hr_hbm.at[row_v.at[0]], ga, sema)

    def pair_body(i, _):
      ja = 2 * i
      pltpu.async_copy(hr_hbm.at[row_v.at[ja + 1]], gb, semb)
      pltpu.make_async_copy(hr_hbm.at[row_v.at[ja]], ga, sema).wait()
      pltpu.sync_copy(ga, agg_sh.at[col_v.at[ja]], add=True)
      pltpu.async_copy(hr_hbm.at[row_v.at[ja + 2]], ga, sema)
      pltpu.make_async_copy(hr_hbm.at[row_v.at[ja + 1]], gb, semb).wait()
      pltpu.sync_copy(gb, agg_sh.at[col_v.at[ja + 1]], add=True)
      return ()
    lax.fori_loop(0, (grp - 1) // 2, pair_body, ())

    pltpu.make_async_copy(hr_hbm.at[row_v.at[grp - 1]], ga, sema).wait()
    pltpu.sync_copy(ga, agg_sh.at[col_v.at[grp - 1]], add=True)
    return ()
  lax.fori_loop(0, ngrp, group_body, ())

  plsc.subcore_barrier()
  pltpu.sync_copy(agg_sh.at[pl.ds(myrow, rpt)],
                  out_hbm.at[cid, pl.ds(myrow, rpt)])


def _sc_agg(hr, row4, col4):
  n, d = hr.shape
  n2 = -(-n // (NS * 128)) * (NS * 128)
  rpt = n2 // NS
  mesh = plsc.VectorSubcoreMesh(core_axis_name="c", subcore_axis_name="s",
                                num_cores=NC, num_subcores=NS)
  body = functools.partial(_sc_agg_body, rpt)
  fn = pl.kernel(
      body,
      out_type=jax.ShapeDtypeStruct((NC, n2, d), jnp.float32),
      mesh=mesh,
      scratch_types=[
          pltpu.VMEM((GRP, CH), jnp.int32),
          pltpu.VMEM((GRP, CH), jnp.int32),
          pltpu.VMEM((CH, d), jnp.float32),
          pltpu.VMEM((CH, d), jnp.float32),
          pltpu.VMEM_SHARED((n2, d), jnp.float32),
          pltpu.SemaphoreType.DMA,
          pltpu.SemaphoreType.DMA,
      ],
  )
  return fn(hr, row4, col4)


def _k0_body(x_ref, w_ref, b_ref, h_ref, hr_ref):
  h = jnp.dot(x_ref[...], w_ref[...],
              preferred_element_type=jnp.float32) + b_ref[...]
  h_ref[...] = h
  hr_ref[...] = jnp.maximum(h, 0.0)


def _k1_body(ope_ref, h_ref, agg_ref, w_ref, b_ref, u_ref, s_ref):
  t = ope_ref[...] * h_ref[...] + agg_ref[0] + agg_ref[1]
  u = jnp.dot(t, w_ref[...], preferred_element_type=jnp.float32) + b_ref[...]
  u_ref[...] = u
  s = jnp.concatenate([jnp.sum(u, 0, keepdims=True),
                       jnp.sum(u * u, 0, keepdims=True)], axis=0)

  @pl.when(pl.program_id(0) == 0)
  def _():
    s_ref[...] = s

  @pl.when(pl.program_id(0) != 0)
  def _():
    s_ref[...] = s_ref[...] + s


def _k2_body(n_rows, u_ref, s_ref, g_ref, be_ref, w_ref, b_ref, v_ref, s2_ref):
  mean = s_ref[0:1, :] * (1.0 / n_rows)
  var = s_ref[1:2, :] * (1.0 / n_rows) - mean * mean
  scale = g_ref[...] * lax.rsqrt(var + 1e-5)
  shift = be_ref[...] - mean * scale
  un = jnp.maximum(u_ref[...] * scale + shift, 0.0)
  v = jnp.dot(un, w_ref[...], preferred_element_type=jnp.float32) + b_ref[...]
  v_ref[...] = v
  s2 = jnp.concatenate([jnp.sum(v, 0, keepdims=True),
                        jnp.sum(v * v, 0, keepdims=True)], axis=0)

  @pl.when(pl.program_id(0) == 0)
  def _():
    s2_ref[...] = s2

  @pl.when(pl.program_id(0) != 0)
  def _():
    s2_ref[...] = s2_ref[...] + s2


def _k3_body(do_relu, n_rows, v_ref, s_ref, g_ref, be_ref, h_ref):
  mean = s_ref[0:1, :] * (1.0 / n_rows)
  var = s_ref[1:2, :] * (1.0 / n_rows) - mean * mean
  scale = g_ref[...] * lax.rsqrt(var + 1e-5)
  shift = be_ref[...] - mean * scale
  h = v_ref[...] * scale + shift
  if do_relu:
    h = jnp.maximum(h, 0.0)
  h_ref[...] = h


_SEQ = pltpu.CompilerParams(dimension_semantics=("arbitrary",))


def _row_block(bn, d):
  return pl.BlockSpec((bn, d), lambda i: (i, 0))


def _full(shape):
  return pl.BlockSpec(shape, lambda i: tuple(0 for _ in shape))


def _k0(x, wt, b, bn):
  n, d = x.shape
  return pl.pallas_call(
      _k0_body,
      grid=(n // bn,),
      in_specs=[_row_block(bn, d), _full(wt.shape), _full(b.shape)],
      out_specs=[_row_block(bn, d), _row_block(bn, d)],
      out_shape=[jax.ShapeDtypeStruct((n, d), jnp.float32)] * 2,
      compiler_params=_SEQ,
  )(x, wt, b)


def _k1(ope, h, agg, wt, b, bn):
  n, d = h.shape
  d2 = wt.shape[1]
  return pl.pallas_call(
      _k1_body,
      grid=(n // bn,),
      in_specs=[_full(ope.shape), _row_block(bn, d),
                pl.BlockSpec((2, bn, d), lambda i: (0, i, 0)),
                _full(wt.shape), _full(b.shape)],
      out_specs=[_row_block(bn, d2), _full((2, d2))],
      out_shape=[jax.ShapeDtypeStruct((n, d2), jnp.float32),
                 jax.ShapeDtypeStruct((2, d2), jnp.float32)],
      compiler_params=_SEQ,
  )(ope, h, agg, wt, b)


def _k2(u, s, g, be, wt, b, bn):
  n, d2 = u.shape
  d = wt.shape[1]
  return pl.pallas_call(
      functools.partial(_k2_body, float(n)),
      grid=(n // bn,),
      in_specs=[_row_block(bn, d2), _full((2, d2)), _full(g.shape),
                _full(be.shape), _full(wt.shape), _full(b.shape)],
      out_specs=[_row_block(bn, d), _full((2, d))],
      out_shape=[jax.ShapeDtypeStruct((n, d), jnp.float32),
                 jax.ShapeDtypeStruct((2, d), jnp.float32)],
      compiler_params=_SEQ,
  )(u, s, g, be, wt, b)


def _k3(v, s, g, be, do_relu, bn):
  n, d = v.shape
  return pl.pallas_call(
      functools.partial(_k3_body, do_relu, float(n)),
      grid=(n // bn,),
      in_specs=[_row_block(bn, d), _full((2, d)), _full(g.shape),
                _full(be.shape)],
      out_specs=_row_block(bn, d),
      out_shape=jax.ShapeDtypeStruct((n, d), jnp.float32),
      compiler_params=_SEQ,
  )(v, s, g, be)


def kernel(x, edge_index, fc_w, fc_b, eps, W1, b1, g1, be1, W2, b2, gno, bno):
  n, d = x.shape
  e = edge_index.shape[1]
  nl = W1.shape[0]
  bn = 1000 if n % 1000 == 0 else n

  epw = e // NW
  gsz = GRP * CH
  epw_pad = -(-epw // gsz) * gsz
  ngrp = epw_pad // gsz
  pad = epw_pad - epw
  row2 = edge_index[0].reshape(NW, epw)
  col2 = edge_index[1].reshape(NW, epw)
  if pad:
    n2 = -(-n // (NS * 128)) * (NS * 128)
    padcol = n + (jnp.arange(pad, dtype=jnp.int32) % (n2 - n))
    row2 = jnp.concatenate([row2, jnp.zeros((NW, pad), jnp.int32)], axis=1)
    col2 = jnp.concatenate(
        [col2, jnp.broadcast_to(padcol, (NW, pad))], axis=1)
  row4 = row2.reshape(NW, ngrp, GRP, CH)
  col4 = col2.reshape(NW, ngrp, GRP, CH)

  h, hr = _k0(x, fc_w.T, fc_b.reshape(1, d), bn)
  for l in range(nl):
    agg = _sc_agg(hr if l == 0 else h, row4, col4)
    ope = jnp.broadcast_to(1.0 + eps[l], (1, d))
    u, s1 = _k1(ope, h, agg, W1[l].T, b1[l].reshape(1, -1), bn)
    v, s2 = _k2(u, s1, g1[l].reshape(1, -1), be1[l].reshape(1, -1),
                W2[l].T, b2[l].reshape(1, -1), bn)
    h = _k3(v, s2, gno[l].reshape(1, -1), bno[l].reshape(1, -1), l < nl - 1, bn)
  return h


# speedup vs baseline: 1.4756x; 1.4756x over previous
import functools

import jax
import jax.numpy as jnp
from jax import lax
from jax.experimental import pallas as pl
from jax.experimental.pallas import tpu as pltpu
from jax.experimental.pallas import tpu_sc as plsc

NC = 2
NS = 16
NW = NC * NS
CH = 80
GRP = 25


def _sc_agg_body(rpt, hr_hbm, row_hbm, col_hbm, out_hbm,
                 row_v, col_v, ga, gb, agg_sh, sema, semb):
  cid = lax.axis_index("c")
  sid = lax.axis_index("s")
  wid = sid * NC + cid

  zeros = jnp.zeros((16,), jnp.float32)
  zr, zc = ga.shape

  def zero_body(r, _):
    for c8 in range(zc // 16):
      ga[r, pl.ds(c8 * 16, 16)] = zeros
    return ()
  lax.fori_loop(0, zr, zero_body, ())

  myrow = sid * rpt
  for t in range(rpt // zr):
    pltpu.sync_copy(ga, agg_sh.at[pl.ds(myrow + t * zr, zr)])
  rem = rpt % zr
  if rem:
    pltpu.sync_copy(ga.at[pl.ds(0, rem)],
                    agg_sh.at[pl.ds(myrow + (rpt // zr) * zr, rem)])

  plsc.subcore_barrier()

  ngrp = row_hbm.shape[1]
  grp = row_hbm.shape[2]

  def group_body(g, _):
    pltpu.sync_copy(row_hbm.at[wid, g], row_v)
    pltpu.sync_copy(col_hbm.at[wid, g], col_v)

    pltpu.async_copy(hr_hbm.at[row_v.at[0]], ga, sema)

    def pair_body(i, _):
      ja = 2 * i
      pltpu.async_copy(hr_hbm.at[row_v.at[ja + 1]], gb, semb)
      pltpu.make_async_copy(hr_hbm.at[row_v.at[ja]], ga, sema).wait()
      pltpu.sync_copy(ga, agg_sh.at[col_v.at[ja]], add=True)
      pltpu.async_copy(hr_hbm.at[row_v.at[ja + 2]], ga, sema)
      pltpu.make_async_copy(hr_hbm.at[row_v.at[ja + 1]], gb, semb).wait()
      pltpu.sync_copy(gb, agg_sh.at[col_v.at[ja + 1]], add=True)
      return ()
    lax.fori_loop(0, (grp - 1) // 2, pair_body, ())

    pltpu.make_async_copy(hr_hbm.at[row_v.at[grp - 1]], ga, sema).wait()
    pltpu.sync_copy(ga, agg_sh.at[col_v.at[grp - 1]], add=True)
    return ()
  lax.fori_loop(0, ngrp, group_body, ())

  plsc.subcore_barrier()
  pltpu.sync_copy(agg_sh.at[pl.ds(myrow, rpt)],
                  out_hbm.at[cid, pl.ds(myrow, rpt)])


def _sc_agg(hr, row4, col4):
  n, d = hr.shape
  n2 = -(-n // (NS * 128)) * (NS * 128)
  rpt = n2 // NS
  mesh = plsc.VectorSubcoreMesh(core_axis_name="c", subcore_axis_name="s",
                                num_cores=NC, num_subcores=NS)
  body = functools.partial(_sc_agg_body, rpt)
  fn = pl.kernel(
      body,
      out_type=jax.ShapeDtypeStruct((NC, n2, d), jnp.float32),
      mesh=mesh,
      scratch_types=[
          pltpu.VMEM((GRP, CH), jnp.int32),
          pltpu.VMEM((GRP, CH), jnp.int32),
          pltpu.VMEM((CH, d), jnp.float32),
          pltpu.VMEM((CH, d), jnp.float32),
          pltpu.VMEM_SHARED((n2, d), jnp.float32),
          pltpu.SemaphoreType.DMA,
          pltpu.SemaphoreType.DMA,
      ],
  )
  return fn(hr, row4, col4)


def _k0_body(x_ref, w_ref, b_ref, h_ref, hr_ref):
  h = jnp.dot(x_ref[...], w_ref[...],
              preferred_element_type=jnp.float32) + b_ref[...]
  h_ref[...] = h
  hr_ref[...] = jnp.maximum(h, 0.0)


def _k1_body(ope_ref, h_ref, agg_ref, w_ref, b_ref, u_ref, s_ref):
  t = ope_ref[...] * h_ref[...] + agg_ref[0] + agg_ref[1]
  u = jnp.dot(t, w_ref[...], preferred_element_type=jnp.float32) + b_ref[...]
  u_ref[...] = u
  s = jnp.concatenate([jnp.sum(u, 0, keepdims=True),
                       jnp.sum(u * u, 0, keepdims=True)], axis=0)

  @pl.when(pl.program_id(0) == 0)
  def _():
    s_ref[...] = s

  @pl.when(pl.program_id(0) != 0)
  def _():
    s_ref[...] = s_ref[...] + s


def _k2_body(n_rows, u_ref, s_ref, g_ref, be_ref, w_ref, b_ref, v_ref, s2_ref):
  mean = s_ref[0:1, :] * (1.0 / n_rows)
  var = s_ref[1:2, :] * (1.0 / n_rows) - mean * mean
  scale = g_ref[...] * lax.rsqrt(var + 1e-5)
  shift = be_ref[...] - mean * scale
  un = jnp.maximum(u_ref[...] * scale + shift, 0.0)
  v = jnp.dot(un, w_ref[...], preferred_element_type=jnp.float32) + b_ref[...]
  v_ref[...] = v
  s2 = jnp.concatenate([jnp.sum(v, 0, keepdims=True),
                        jnp.sum(v * v, 0, keepdims=True)], axis=0)

  @pl.when(pl.program_id(0) == 0)
  def _():
    s2_ref[...] = s2

  @pl.when(pl.program_id(0) != 0)
  def _():
    s2_ref[...] = s2_ref[...] + s2


def _k3_body(do_relu, n_rows, v_ref, s_ref, g_ref, be_ref, h_ref):
  mean = s_ref[0:1, :] * (1.0 / n_rows)
  var = s_ref[1:2, :] * (1.0 / n_rows) - mean * mean
  scale = g_ref[...] * lax.rsqrt(var + 1e-5)
  shift = be_ref[...] - mean * scale
  h = v_ref[...] * scale + shift
  if do_relu:
    h = jnp.maximum(h, 0.0)
  h_ref[...] = h


_SEQ = pltpu.CompilerParams(dimension_semantics=("arbitrary",))


def _row_block(bn, d):
  return pl.BlockSpec((bn, d), lambda i: (i, 0))


def _full(shape):
  return pl.BlockSpec(shape, lambda i: tuple(0 for _ in shape))


def _k0(x, wt, b, bn):
  n, d = x.shape
  return pl.pallas_call(
      _k0_body,
      grid=(n // bn,),
      in_specs=[_row_block(bn, d), _full(wt.shape), _full(b.shape)],
      out_specs=[_row_block(bn, d), _row_block(bn, d)],
      out_shape=[jax.ShapeDtypeStruct((n, d), jnp.float32)] * 2,
      compiler_params=_SEQ,
  )(x, wt, b)


def _k1(ope, h, agg, wt, b, bn):
  n, d = h.shape
  d2 = wt.shape[1]
  return pl.pallas_call(
      _k1_body,
      grid=(n // bn,),
      in_specs=[_full(ope.shape), _row_block(bn, d),
                pl.BlockSpec((2, bn, d), lambda i: (0, i, 0)),
                _full(wt.shape), _full(b.shape)],
      out_specs=[_row_block(bn, d2), _full((2, d2))],
      out_shape=[jax.ShapeDtypeStruct((n, d2), jnp.float32),
                 jax.ShapeDtypeStruct((2, d2), jnp.float32)],
      compiler_params=_SEQ,
  )(ope, h, agg, wt, b)


def _k2(u, s, g, be, wt, b, bn):
  n, d2 = u.shape
  d = wt.shape[1]
  return pl.pallas_call(
      functools.partial(_k2_body, float(n)),
      grid=(n // bn,),
      in_specs=[_row_block(bn, d2), _full((2, d2)), _full(g.shape),
                _full(be.shape), _full(wt.shape), _full(b.shape)],
      out_specs=[_row_block(bn, d), _full((2, d))],
      out_shape=[jax.ShapeDtypeStruct((n, d), jnp.float32),
                 jax.ShapeDtypeStruct((2, d), jnp.float32)],
      compiler_params=_SEQ,
  )(u, s, g, be, wt, b)


def _k3(v, s, g, be, do_relu, bn):
  n, d = v.shape
  return pl.pallas_call(
      functools.partial(_k3_body, do_relu, float(n)),
      grid=(n // bn,),
      in_specs=[_row_block(bn, d), _full((2, d)), _full(g.shape),
                _full(be.shape)],
      out_specs=_row_block(bn, d),
      out_shape=jax.ShapeDtypeStruct((n, d), jnp.float32),
      compiler_params=_SEQ,
  )(v, s, g, be)


def kernel(x, edge_index, fc_w, fc_b, eps, W1, b1, g1, be1, W2, b2, gno, bno):
  n, d = x.shape
  e = edge_index.shape[1]
  nl = W1.shape[0]
  bn = 1000 if n % 1000 == 0 else n

  epw = e // NW
  gsz = GRP * CH
  epw_pad = -(-epw // gsz) * gsz
  ngrp = epw_pad // gsz
  pad = epw_pad - epw
  row2 = edge_index[0].reshape(NW, epw)
  col2 = edge_index[1].reshape(NW, epw)
  if pad:
    n2 = -(-n // (NS * 128)) * (NS * 128)
    padcol = n + (jnp.arange(pad, dtype=jnp.int32) % (n2 - n))
    row2 = jnp.concatenate([row2, jnp.zeros((NW, pad), jnp.int32)], axis=1)
    col2 = jnp.concatenate(
        [col2, jnp.broadcast_to(padcol, (NW, pad))], axis=1)
  row4 = row2.reshape(NW, ngrp, GRP, CH)
  col4 = col2.reshape(NW, ngrp, GRP, CH)

  h, hr = _k0(x, fc_w.T, fc_b.reshape(1, d), bn)
  for l in range(nl):
    agg = _sc_agg(hr if l == 0 else h, row4, col4)
    ope = jnp.broadcast_to(1.0 + eps[l], (1, d))
    u, s1 = _k1(ope, h, agg, W1[l].T, b1[l].reshape(1, -1), bn)
    v, s2 = _k2(u, s1, g1[l].reshape(1, -1), be1[l].reshape(1, -1),
                W2[l].T, b2[l].reshape(1, -1), bn)
    h = _k3(v, s2, gno[l].reshape(1, -1), bno[l].reshape(1, -1), l < nl - 1, bn)
  return h


# speedup vs baseline: 1.6544x; 1.1212x over previous
import functools

import jax
import jax.numpy as jnp
from jax import lax
from jax.experimental import pallas as pl
from jax.experimental.pallas import tpu as pltpu
from jax.experimental.pallas import tpu_sc as plsc

NC = 2
NS = 16
NW = NC * NS
CH = 80
GRP = 25


def _sc_agg_body(rpt, hr_hbm, row_hbm, col_hbm, out_hbm,
                 row_v, col_v, ga, gb, gc, agg_sh, sema, semb, semc):
  cid = lax.axis_index("c")
  sid = lax.axis_index("s")
  wid = sid * NC + cid
  bufs = (ga, gb, gc)
  sems = (sema, semb, semc)

  def fire(b, j):
    pltpu.async_copy(hr_hbm.at[row_v.at[j]], bufs[b], sems[b])

  def wait_scatter(b, j):
    pltpu.make_async_copy(hr_hbm.at[row_v.at[0]], bufs[b], sems[b]).wait()
    pltpu.sync_copy(bufs[b], agg_sh.at[col_v.at[j]], add=True)

  zeros = jnp.zeros((16,), jnp.float32)
  zr, zc = ga.shape

  def zero_body(r, _):
    for c8 in range(zc // 16):
      ga[r, pl.ds(c8 * 16, 16)] = zeros
    return ()
  lax.fori_loop(0, zr, zero_body, ())

  myrow = sid * rpt
  for t in range(rpt // zr):
    pltpu.sync_copy(ga, agg_sh.at[pl.ds(myrow + t * zr, zr)])
  rem = rpt % zr
  if rem:
    pltpu.sync_copy(ga.at[pl.ds(0, rem)],
                    agg_sh.at[pl.ds(myrow + (rpt // zr) * zr, rem)])

  plsc.subcore_barrier()

  ngrp = row_hbm.shape[1]
  grp = row_hbm.shape[2]

  def group_body(g, _):
    pltpu.sync_copy(row_hbm.at[wid, g], row_v)
    pltpu.sync_copy(col_hbm.at[wid, g], col_v)

    for b in range(3):
      fire(b, b)

    def tri_body(i, _):
      j = 3 * i
      wait_scatter(0, j)
      fire(0, j + 3)
      wait_scatter(1, j + 1)
      fire(1, j + 4)
      wait_scatter(2, j + 2)
      fire(2, j + 5)
      return ()
    lax.fori_loop(0, (grp - 4) // 3, tri_body, ())

    wait_scatter(0, grp - 4)
    fire(0, grp - 1)
    wait_scatter(1, grp - 3)
    wait_scatter(2, grp - 2)
    wait_scatter(0, grp - 1)
    return ()
  lax.fori_loop(0, ngrp, group_body, ())

  plsc.subcore_barrier()
  pltpu.sync_copy(agg_sh.at[pl.ds(myrow, rpt)],
                  out_hbm.at[cid, pl.ds(myrow, rpt)])


def _sc_agg(hr, row4, col4):
  n, d = hr.shape
  n2 = -(-n // (NS * 128)) * (NS * 128)
  rpt = n2 // NS
  mesh = plsc.VectorSubcoreMesh(core_axis_name="c", subcore_axis_name="s",
                                num_cores=NC, num_subcores=NS)
  body = functools.partial(_sc_agg_body, rpt)
  fn = pl.kernel(
      body,
      out_type=jax.ShapeDtypeStruct((NC, n2, d), jnp.float32),
      mesh=mesh,
      scratch_types=[
          pltpu.VMEM((GRP, CH), jnp.int32),
          pltpu.VMEM((GRP, CH), jnp.int32),
          pltpu.VMEM((CH, d), jnp.float32),
          pltpu.VMEM((CH, d), jnp.float32),
          pltpu.VMEM((CH, d), jnp.float32),
          pltpu.VMEM_SHARED((n2, d), jnp.float32),
          pltpu.SemaphoreType.DMA,
          pltpu.SemaphoreType.DMA,
          pltpu.SemaphoreType.DMA,
      ],
  )
  return fn(hr, row4, col4)


def _k0_body(x_ref, w_ref, b_ref, h_ref, hr_ref):
  h = jnp.dot(x_ref[...], w_ref[...],
              preferred_element_type=jnp.float32) + b_ref[...]
  h_ref[...] = h
  hr_ref[...] = jnp.maximum(h, 0.0)


def _k1_body(ope_ref, h_ref, agg_ref, w_ref, b_ref, u_ref, s_ref):
  t = ope_ref[...] * h_ref[...] + agg_ref[0] + agg_ref[1]
  u = jnp.dot(t, w_ref[...], preferred_element_type=jnp.float32) + b_ref[...]
  u_ref[...] = u
  s = jnp.concatenate([jnp.sum(u, 0, keepdims=True),
                       jnp.sum(u * u, 0, keepdims=True)], axis=0)

  @pl.when(pl.program_id(0) == 0)
  def _():
    s_ref[...] = s

  @pl.when(pl.program_id(0) != 0)
  def _():
    s_ref[...] = s_ref[...] + s


def _k2_body(n_rows, u_ref, s_ref, g_ref, be_ref, w_ref, b_ref, v_ref, s2_ref):
  mean = s_ref[0:1, :] * (1.0 / n_rows)
  var = s_ref[1:2, :] * (1.0 / n_rows) - mean * mean
  scale = g_ref[...] * lax.rsqrt(var + 1e-5)
  shift = be_ref[...] - mean * scale
  un = jnp.maximum(u_ref[...] * scale + shift, 0.0)
  v = jnp.dot(un, w_ref[...], preferred_element_type=jnp.float32) + b_ref[...]
  v_ref[...] = v
  s2 = jnp.concatenate([jnp.sum(v, 0, keepdims=True),
                        jnp.sum(v * v, 0, keepdims=True)], axis=0)

  @pl.when(pl.program_id(0) == 0)
  def _():
    s2_ref[...] = s2

  @pl.when(pl.program_id(0) != 0)
  def _():
    s2_ref[...] = s2_ref[...] + s2


def _k3_body(do_relu, n_rows, v_ref, s_ref, g_ref, be_ref, h_ref):
  mean = s_ref[0:1, :] * (1.0 / n_rows)
  var = s_ref[1:2, :] * (1.0 / n_rows) - mean * mean
  scale = g_ref[...] * lax.rsqrt(var + 1e-5)
  shift = be_ref[...] - mean * scale
  h = v_ref[...] * scale + shift
  if do_relu:
    h = jnp.maximum(h, 0.0)
  h_ref[...] = h


_SEQ = pltpu.CompilerParams(dimension_semantics=("arbitrary",))


def _row_block(bn, d):
  return pl.BlockSpec((bn, d), lambda i: (i, 0))


def _full(shape):
  return pl.BlockSpec(shape, lambda i: tuple(0 for _ in shape))


def _k0(x, wt, b, bn):
  n, d = x.shape
  return pl.pallas_call(
      _k0_body,
      grid=(n // bn,),
      in_specs=[_row_block(bn, d), _full(wt.shape), _full(b.shape)],
      out_specs=[_row_block(bn, d), _row_block(bn, d)],
      out_shape=[jax.ShapeDtypeStruct((n, d), jnp.float32)] * 2,
      compiler_params=_SEQ,
  )(x, wt, b)


def _k1(ope, h, agg, wt, b, bn):
  n, d = h.shape
  d2 = wt.shape[1]
  return pl.pallas_call(
      _k1_body,
      grid=(n // bn,),
      in_specs=[_full(ope.shape), _row_block(bn, d),
                pl.BlockSpec((2, bn, d), lambda i: (0, i, 0)),
                _full(wt.shape), _full(b.shape)],
      out_specs=[_row_block(bn, d2), _full((2, d2))],
      out_shape=[jax.ShapeDtypeStruct((n, d2), jnp.float32),
                 jax.ShapeDtypeStruct((2, d2), jnp.float32)],
      compiler_params=_SEQ,
  )(ope, h, agg, wt, b)


def _k2(u, s, g, be, wt, b, bn):
  n, d2 = u.shape
  d = wt.shape[1]
  return pl.pallas_call(
      functools.partial(_k2_body, float(n)),
      grid=(n // bn,),
      in_specs=[_row_block(bn, d2), _full((2, d2)), _full(g.shape),
                _full(be.shape), _full(wt.shape), _full(b.shape)],
      out_specs=[_row_block(bn, d), _full((2, d))],
      out_shape=[jax.ShapeDtypeStruct((n, d), jnp.float32),
                 jax.ShapeDtypeStruct((2, d), jnp.float32)],
      compiler_params=_SEQ,
  )(u, s, g, be, wt, b)


def _k3(v, s, g, be, do_relu, bn):
  n, d = v.shape
  return pl.pallas_call(
      functools.partial(_k3_body, do_relu, float(n)),
      grid=(n // bn,),
      in_specs=[_row_block(bn, d), _full((2, d)), _full(g.shape),
                _full(be.shape)],
      out_specs=_row_block(bn, d),
      out_shape=jax.ShapeDtypeStruct((n, d), jnp.float32),
      compiler_params=_SEQ,
  )(v, s, g, be)


def kernel(x, edge_index, fc_w, fc_b, eps, W1, b1, g1, be1, W2, b2, gno, bno):
  n, d = x.shape
  e = edge_index.shape[1]
  nl = W1.shape[0]
  bn = 1000 if n % 1000 == 0 else n

  epw = e // NW
  gsz = GRP * CH
  epw_pad = -(-epw // gsz) * gsz
  ngrp = epw_pad // gsz
  pad = epw_pad - epw
  row2 = edge_index[0].reshape(NW, epw)
  col2 = edge_index[1].reshape(NW, epw)
  if pad:
    n2 = -(-n // (NS * 128)) * (NS * 128)
    padcol = n + (jnp.arange(pad, dtype=jnp.int32) % (n2 - n))
    row2 = jnp.concatenate([row2, jnp.zeros((NW, pad), jnp.int32)], axis=1)
    col2 = jnp.concatenate(
        [col2, jnp.broadcast_to(padcol, (NW, pad))], axis=1)
  row4 = row2.reshape(NW, ngrp, GRP, CH)
  col4 = col2.reshape(NW, ngrp, GRP, CH)

  h, hr = _k0(x, fc_w.T, fc_b.reshape(1, d), bn)
  for l in range(nl):
    agg = _sc_agg(hr if l == 0 else h, row4, col4)
    ope = jnp.broadcast_to(1.0 + eps[l], (1, d))
    u, s1 = _k1(ope, h, agg, W1[l].T, b1[l].reshape(1, -1), bn)
    v, s2 = _k2(u, s1, g1[l].reshape(1, -1), be1[l].reshape(1, -1),
                W2[l].T, b2[l].reshape(1, -1), bn)
    h = _k3(v, s2, gno[l].reshape(1, -1), bno[l].reshape(1, -1), l < nl - 1, bn)
  return h


# speedup vs baseline: 1.7284x; 1.0447x over previous
import functools

import jax
import jax.numpy as jnp
from jax import lax
from jax.experimental import pallas as pl
from jax.experimental.pallas import tpu as pltpu
from jax.experimental.pallas import tpu_sc as plsc

NC = 2
NS = 16
NW = NC * NS
CH = 80
GRP = 25


def _sc_agg_body(rpt, hr_hbm, row_hbm, col_hbm, out_hbm,
                 row_v, col_v, ga, gb, gc, agg_sh, sema, semb, semc):
  cid = lax.axis_index("c")
  sid = lax.axis_index("s")
  wid = sid * NC + cid
  bufs = (ga, gb, gc)
  sems = (sema, semb, semc)

  def fire(b, j):
    pltpu.async_copy(hr_hbm.at[row_v.at[j]], bufs[b], sems[b])

  def wait_scatter(b, j):
    pltpu.make_async_copy(hr_hbm.at[row_v.at[0]], bufs[b], sems[b]).wait()
    pltpu.sync_copy(bufs[b], agg_sh.at[col_v.at[j]], add=True)

  zeros = jnp.zeros((16,), jnp.float32)
  zr, zc = ga.shape

  def zero_body(r, _):
    for c8 in range(zc // 16):
      ga[r, pl.ds(c8 * 16, 16)] = zeros
    return ()
  lax.fori_loop(0, zr, zero_body, ())

  myrow = sid * rpt
  for t in range(rpt // zr):
    pltpu.sync_copy(ga, agg_sh.at[pl.ds(myrow + t * zr, zr)])
  rem = rpt % zr
  if rem:
    pltpu.sync_copy(ga.at[pl.ds(0, rem)],
                    agg_sh.at[pl.ds(myrow + (rpt // zr) * zr, rem)])

  plsc.subcore_barrier()

  ngrp = row_hbm.shape[1]
  grp = row_hbm.shape[2]

  def group_body(g, _):
    pltpu.sync_copy(row_hbm.at[wid, g], row_v)
    pltpu.sync_copy(col_hbm.at[wid, g], col_v)

    for b in range(3):
      fire(b, b)

    def tri_body(i, _):
      j = 3 * i
      wait_scatter(0, j)
      fire(0, j + 3)
      wait_scatter(1, j + 1)
      fire(1, j + 4)
      wait_scatter(2, j + 2)
      fire(2, j + 5)
      return ()
    lax.fori_loop(0, (grp - 4) // 3, tri_body, ())

    wait_scatter(0, grp - 4)
    fire(0, grp - 1)
    wait_scatter(1, grp - 3)
    wait_scatter(2, grp - 2)
    wait_scatter(0, grp - 1)
    return ()
  lax.fori_loop(0, ngrp, group_body, ())

  plsc.subcore_barrier()
  pltpu.sync_copy(agg_sh.at[pl.ds(myrow, rpt)],
                  out_hbm.at[cid, pl.ds(myrow, rpt)])


def _sc_agg(hr, row4, col4):
  n, d = hr.shape
  n2 = -(-n // (NS * 128)) * (NS * 128)
  rpt = n2 // NS
  mesh = plsc.VectorSubcoreMesh(core_axis_name="c", subcore_axis_name="s",
                                num_cores=NC, num_subcores=NS)
  body = functools.partial(_sc_agg_body, rpt)
  fn = pl.kernel(
      body,
      out_type=jax.ShapeDtypeStruct((NC, n2, d), jnp.float32),
      mesh=mesh,
      scratch_types=[
          pltpu.VMEM((GRP, CH), jnp.int32),
          pltpu.VMEM((GRP, CH), jnp.int32),
          pltpu.VMEM((CH, d), jnp.float32),
          pltpu.VMEM((CH, d), jnp.float32),
          pltpu.VMEM((CH, d), jnp.float32),
          pltpu.VMEM_SHARED((n2, d), jnp.float32),
          pltpu.SemaphoreType.DMA,
          pltpu.SemaphoreType.DMA,
          pltpu.SemaphoreType.DMA,
      ],
  )
  return fn(hr, row4, col4)


def _k0_body(x_ref, w_ref, b_ref, h_ref, hr_ref):
  h = jnp.dot(x_ref[...], w_ref[...],
              preferred_element_type=jnp.float32) + b_ref[...]
  h_ref[...] = h
  hr_ref[...] = jnp.maximum(h, 0.0)


def _k1_body(ope_ref, h_ref, agg_ref, w_ref, b_ref, u_ref, s_ref):
  t = ope_ref[...] * h_ref[...] + agg_ref[0] + agg_ref[1]
  u = jnp.dot(t, w_ref[...], preferred_element_type=jnp.float32) + b_ref[...]
  u_ref[...] = u
  s = jnp.concatenate([jnp.sum(u, 0, keepdims=True),
                       jnp.sum(u * u, 0, keepdims=True)], axis=0)

  @pl.when(pl.program_id(0) == 0)
  def _():
    s_ref[...] = s

  @pl.when(pl.program_id(0) != 0)
  def _():
    s_ref[...] = s_ref[...] + s


def _k2_body(n_rows, u_ref, s_ref, g_ref, be_ref, w_ref, b_ref, v_ref, s2_ref):
  mean = s_ref[0:1, :] * (1.0 / n_rows)
  var = s_ref[1:2, :] * (1.0 / n_rows) - mean * mean
  scale = g_ref[...] * lax.rsqrt(var + 1e-5)
  shift = be_ref[...] - mean * scale
  un = jnp.maximum(u_ref[...] * scale + shift, 0.0)
  v = jnp.dot(un, w_ref[...], preferred_element_type=jnp.float32) + b_ref[...]
  v_ref[...] = v
  s2 = jnp.concatenate([jnp.sum(v, 0, keepdims=True),
                        jnp.sum(v * v, 0, keepdims=True)], axis=0)

  @pl.when(pl.program_id(0) == 0)
  def _():
    s2_ref[...] = s2

  @pl.when(pl.program_id(0) != 0)
  def _():
    s2_ref[...] = s2_ref[...] + s2


def _k3_body(do_relu, n_rows, v_ref, s_ref, g_ref, be_ref, h_ref):
  mean = s_ref[0:1, :] * (1.0 / n_rows)
  var = s_ref[1:2, :] * (1.0 / n_rows) - mean * mean
  scale = g_ref[...] * lax.rsqrt(var + 1e-5)
  shift = be_ref[...] - mean * scale
  h = v_ref[...] * scale + shift
  if do_relu:
    h = jnp.maximum(h, 0.0)
  h_ref[...] = h


_SEQ = pltpu.CompilerParams(dimension_semantics=("arbitrary",))


def _row_block(bn, d):
  return pl.BlockSpec((bn, d), lambda i: (i, 0))


def _full(shape):
  return pl.BlockSpec(shape, lambda i: tuple(0 for _ in shape))


def _k0(x, wt, b, bn):
  n, d = x.shape
  return pl.pallas_call(
      _k0_body,
      grid=(n // bn,),
      in_specs=[_row_block(bn, d), _full(wt.shape), _full(b.shape)],
      out_specs=[_row_block(bn, d), _row_block(bn, d)],
      out_shape=[jax.ShapeDtypeStruct((n, d), jnp.float32)] * 2,
      compiler_params=_SEQ,
  )(x, wt, b)


def _k1(ope, h, agg, wt, b, bn):
  n, d = h.shape
  d2 = wt.shape[1]
  return pl.pallas_call(
      _k1_body,
      grid=(n // bn,),
      in_specs=[_full(ope.shape), _row_block(bn, d),
                pl.BlockSpec((2, bn, d), lambda i: (0, i, 0)),
                _full(wt.shape), _full(b.shape)],
      out_specs=[_row_block(bn, d2), _full((2, d2))],
      out_shape=[jax.ShapeDtypeStruct((n, d2), jnp.float32),
                 jax.ShapeDtypeStruct((2, d2), jnp.float32)],
      compiler_params=_SEQ,
  )(ope, h, agg, wt, b)


def _k2(u, s, g, be, wt, b, bn):
  n, d2 = u.shape
  d = wt.shape[1]
  return pl.pallas_call(
      functools.partial(_k2_body, float(n)),
      grid=(n // bn,),
      in_specs=[_row_block(bn, d2), _full((2, d2)), _full(g.shape),
                _full(be.shape), _full(wt.shape), _full(b.shape)],
      out_specs=[_row_block(bn, d), _full((2, d))],
      out_shape=[jax.ShapeDtypeStruct((n, d), jnp.float32),
                 jax.ShapeDtypeStruct((2, d), jnp.float32)],
      compiler_params=_SEQ,
  )(u, s, g, be, wt, b)


def _k3(v, s, g, be, do_relu, bn):
  n, d = v.shape
  return pl.pallas_call(
      functools.partial(_k3_body, do_relu, float(n)),
      grid=(n // bn,),
      in_specs=[_row_block(bn, d), _full((2, d)), _full(g.shape),
                _full(be.shape)],
      out_specs=_row_block(bn, d),
      out_shape=jax.ShapeDtypeStruct((n, d), jnp.float32),
      compiler_params=_SEQ,
  )(v, s, g, be)


def kernel(x, edge_index, fc_w, fc_b, eps, W1, b1, g1, be1, W2, b2, gno, bno):
  n, d = x.shape
  e = edge_index.shape[1]
  nl = W1.shape[0]
  bn = 2000 if n % 2000 == 0 else n

  epw = e // NW
  gsz = GRP * CH
  epw_pad = -(-epw // gsz) * gsz
  ngrp = epw_pad // gsz
  pad = epw_pad - epw
  row2 = edge_index[0].reshape(NW, epw)
  col2 = edge_index[1].reshape(NW, epw)
  if pad:
    n2 = -(-n // (NS * 128)) * (NS * 128)
    padcol = n + (jnp.arange(pad, dtype=jnp.int32) % (n2 - n))
    row2 = jnp.concatenate([row2, jnp.zeros((NW, pad), jnp.int32)], axis=1)
    col2 = jnp.concatenate(
        [col2, jnp.broadcast_to(padcol, (NW, pad))], axis=1)
  row4 = row2.reshape(NW, ngrp, GRP, CH)
  col4 = col2.reshape(NW, ngrp, GRP, CH)

  h, hr = _k0(x, fc_w.T, fc_b.reshape(1, d), bn)
  for l in range(nl):
    agg = _sc_agg(hr if l == 0 else h, row4, col4)
    ope = jnp.broadcast_to(1.0 + eps[l], (1, d))
    u, s1 = _k1(ope, h, agg, W1[l].T, b1[l].reshape(1, -1), bn)
    v, s2 = _k2(u, s1, g1[l].reshape(1, -1), be1[l].reshape(1, -1),
                W2[l].T, b2[l].reshape(1, -1), bn)
    h = _k3(v, s2, gno[l].reshape(1, -1), bno[l].reshape(1, -1), l < nl - 1, bn)
  return h


# speedup vs baseline: 1.8239x; 1.0553x over previous
import functools

import jax
import jax.numpy as jnp
from jax import lax
from jax.experimental import pallas as pl
from jax.experimental.pallas import tpu as pltpu
from jax.experimental.pallas import tpu_sc as plsc

NC = 2
NS = 16
NW = NC * NS
CH = 80
GRP = 25


def _sc_agg_body(rpt, hr_hbm, row_hbm, col_hbm, out_hbm,
                 row_v, col_v, ga, gb, gc, agg_sh, sema, semb, semc):
  cid = lax.axis_index("c")
  sid = lax.axis_index("s")
  wid = sid * NC + cid
  bufs = (ga, gb, gc)
  sems = (sema, semb, semc)

  def fire(b, j):
    pltpu.async_copy(hr_hbm.at[row_v.at[j]], bufs[b], sems[b])

  def wait_scatter(b, j):
    pltpu.make_async_copy(hr_hbm.at[row_v.at[0]], bufs[b], sems[b]).wait()
    pltpu.sync_copy(bufs[b], agg_sh.at[col_v.at[j]], add=True)

  zeros = jnp.zeros((16,), jnp.float32)
  zr, zc = ga.shape

  def zero_body(r, _):
    for c8 in range(zc // 16):
      ga[r, pl.ds(c8 * 16, 16)] = zeros
    return ()
  lax.fori_loop(0, zr, zero_body, ())

  myrow = sid * rpt
  for t in range(rpt // zr):
    pltpu.sync_copy(ga, agg_sh.at[pl.ds(myrow + t * zr, zr)])
  rem = rpt % zr
  if rem:
    pltpu.sync_copy(ga.at[pl.ds(0, rem)],
                    agg_sh.at[pl.ds(myrow + (rpt // zr) * zr, rem)])

  plsc.subcore_barrier()

  ngrp = row_hbm.shape[1]
  grp = row_hbm.shape[2]

  def group_body(g, _):
    pltpu.sync_copy(row_hbm.at[wid, g], row_v)
    pltpu.sync_copy(col_hbm.at[wid, g], col_v)

    for b in range(3):
      fire(b, b)

    def tri_body(i, _):
      j = 3 * i
      wait_scatter(0, j)
      fire(0, j + 3)
      wait_scatter(1, j + 1)
      fire(1, j + 4)
      wait_scatter(2, j + 2)
      fire(2, j + 5)
      return ()
    lax.fori_loop(0, (grp - 4) // 3, tri_body, ())

    wait_scatter(0, grp - 4)
    fire(0, grp - 1)
    wait_scatter(1, grp - 3)
    wait_scatter(2, grp - 2)
    wait_scatter(0, grp - 1)
    return ()
  lax.fori_loop(0, ngrp, group_body, ())

  plsc.subcore_barrier()
  pltpu.sync_copy(agg_sh.at[pl.ds(myrow, rpt)],
                  out_hbm.at[cid, pl.ds(myrow, rpt)])


def _sc_agg(hr, row4, col4):
  n, d = hr.shape
  n2 = -(-n // (NS * 128)) * (NS * 128)
  rpt = n2 // NS
  mesh = plsc.VectorSubcoreMesh(core_axis_name="c", subcore_axis_name="s",
                                num_cores=NC, num_subcores=NS)
  body = functools.partial(_sc_agg_body, rpt)
  fn = pl.kernel(
      body,
      out_type=jax.ShapeDtypeStruct((NC, n2, d), jnp.float32),
      mesh=mesh,
      scratch_types=[
          pltpu.VMEM((GRP, CH), jnp.int32),
          pltpu.VMEM((GRP, CH), jnp.int32),
          pltpu.VMEM((CH, d), jnp.float32),
          pltpu.VMEM((CH, d), jnp.float32),
          pltpu.VMEM((CH, d), jnp.float32),
          pltpu.VMEM_SHARED((n2, d), jnp.float32),
          pltpu.SemaphoreType.DMA,
          pltpu.SemaphoreType.DMA,
          pltpu.SemaphoreType.DMA,
      ],
  )
  return fn(hr, row4, col4)


def _k0_body(x_ref, w_ref, b_ref, h_ref, hr_ref):
  h = jnp.dot(x_ref[...], w_ref[...],
              preferred_element_type=jnp.float32) + b_ref[...]
  h_ref[...] = h
  hr_ref[...] = jnp.maximum(h, 0.0)


def _k1_body(ope_ref, h_ref, agg_ref, w_ref, b_ref, u_ref, s_ref):
  t = ope_ref[...] * h_ref[...] + agg_ref[0] + agg_ref[1]
  u = jnp.dot(t, w_ref[...], preferred_element_type=jnp.float32) + b_ref[...]
  u_ref[...] = u
  s = jnp.concatenate([jnp.sum(u, 0, keepdims=True),
                       jnp.sum(u * u, 0, keepdims=True)], axis=0)

  @pl.when(pl.program_id(0) == 0)
  def _():
    s_ref[...] = s

  @pl.when(pl.program_id(0) != 0)
  def _():
    s_ref[...] = s_ref[...] + s


def _k2_body(n_rows, u_ref, s_ref, g_ref, be_ref, w_ref, b_ref, v_ref, s2_ref):
  mean = s_ref[0:1, :] * (1.0 / n_rows)
  var = s_ref[1:2, :] * (1.0 / n_rows) - mean * mean
  scale = g_ref[...] * lax.rsqrt(var + 1e-5)
  shift = be_ref[...] - mean * scale
  un = jnp.maximum(u_ref[...] * scale + shift, 0.0)
  v = jnp.dot(un, w_ref[...], preferred_element_type=jnp.float32) + b_ref[...]
  v_ref[...] = v
  s2 = jnp.concatenate([jnp.sum(v, 0, keepdims=True),
                        jnp.sum(v * v, 0, keepdims=True)], axis=0)

  @pl.when(pl.program_id(0) == 0)
  def _():
    s2_ref[...] = s2

  @pl.when(pl.program_id(0) != 0)
  def _():
    s2_ref[...] = s2_ref[...] + s2


def _k3_body(do_relu, n_rows, v_ref, s_ref, g_ref, be_ref, h_ref):
  mean = s_ref[0:1, :] * (1.0 / n_rows)
  var = s_ref[1:2, :] * (1.0 / n_rows) - mean * mean
  scale = g_ref[...] * lax.rsqrt(var + 1e-5)
  shift = be_ref[...] - mean * scale
  h = v_ref[...] * scale + shift
  if do_relu:
    h = jnp.maximum(h, 0.0)
  h_ref[...] = h


_SEQ = pltpu.CompilerParams(dimension_semantics=("arbitrary",))


def _row_block(bn, d):
  return pl.BlockSpec((bn, d), lambda i: (i, 0))


def _full(shape):
  return pl.BlockSpec(shape, lambda i: tuple(0 for _ in shape))


def _k0(x, wt, b, bn):
  n, d = x.shape
  return pl.pallas_call(
      _k0_body,
      grid=(n // bn,),
      in_specs=[_row_block(bn, d), _full(wt.shape), _full(b.shape)],
      out_specs=[_row_block(bn, d), _row_block(bn, d)],
      out_shape=[jax.ShapeDtypeStruct((n, d), jnp.float32)] * 2,
      compiler_params=_SEQ,
  )(x, wt, b)


def _k1(ope, h, agg, wt, b, bn):
  n, d = h.shape
  d2 = wt.shape[1]
  return pl.pallas_call(
      _k1_body,
      grid=(n // bn,),
      in_specs=[_full(ope.shape), _row_block(bn, d),
                pl.BlockSpec((2, bn, d), lambda i: (0, i, 0)),
                _full(wt.shape), _full(b.shape)],
      out_specs=[_row_block(bn, d2), _full((2, d2))],
      out_shape=[jax.ShapeDtypeStruct((n, d2), jnp.float32),
                 jax.ShapeDtypeStruct((2, d2), jnp.float32)],
      compiler_params=_SEQ,
  )(ope, h, agg, wt, b)


def _k2(u, s, g, be, wt, b, bn):
  n, d2 = u.shape
  d = wt.shape[1]
  return pl.pallas_call(
      functools.partial(_k2_body, float(n)),
      grid=(n // bn,),
      in_specs=[_row_block(bn, d2), _full((2, d2)), _full(g.shape),
                _full(be.shape), _full(wt.shape), _full(b.shape)],
      out_specs=[_row_block(bn, d), _full((2, d))],
      out_shape=[jax.ShapeDtypeStruct((n, d), jnp.float32),
                 jax.ShapeDtypeStruct((2, d), jnp.float32)],
      compiler_params=_SEQ,
  )(u, s, g, be, wt, b)


def _k3(v, s, g, be, do_relu, bn):
  n, d = v.shape
  return pl.pallas_call(
      functools.partial(_k3_body, do_relu, float(n)),
      grid=(n // bn,),
      in_specs=[_row_block(bn, d), _full((2, d)), _full(g.shape),
                _full(be.shape)],
      out_specs=_row_block(bn, d),
      out_shape=jax.ShapeDtypeStruct((n, d), jnp.float32),
      compiler_params=_SEQ,
  )(v, s, g, be)


def kernel(x, edge_index, fc_w, fc_b, eps, W1, b1, g1, be1, W2, b2, gno, bno):
  n, d = x.shape
  e = edge_index.shape[1]
  nl = W1.shape[0]
  bn = 5000 if n % 5000 == 0 else n

  epw = e // NW
  gsz = GRP * CH
  epw_pad = -(-epw // gsz) * gsz
  ngrp = epw_pad // gsz
  pad = epw_pad - epw
  row2 = edge_index[0].reshape(NW, epw)
  col2 = edge_index[1].reshape(NW, epw)
  if pad:
    n2 = -(-n // (NS * 128)) * (NS * 128)
    padcol = n + (jnp.arange(pad, dtype=jnp.int32) % (n2 - n))
    row2 = jnp.concatenate([row2, jnp.zeros((NW, pad), jnp.int32)], axis=1)
    col2 = jnp.concatenate(
        [col2, jnp.broadcast_to(padcol, (NW, pad))], axis=1)
  row4 = row2.reshape(NW, ngrp, GRP, CH)
  col4 = col2.reshape(NW, ngrp, GRP, CH)

  h, hr = _k0(x, fc_w.T, fc_b.reshape(1, d), bn)
  for l in range(nl):
    agg = _sc_agg(hr if l == 0 else h, row4, col4)
    ope = jnp.broadcast_to(1.0 + eps[l], (1, d))
    u, s1 = _k1(ope, h, agg, W1[l].T, b1[l].reshape(1, -1), bn)
    v, s2 = _k2(u, s1, g1[l].reshape(1, -1), be1[l].reshape(1, -1),
                W2[l].T, b2[l].reshape(1, -1), bn)
    h = _k3(v, s2, gno[l].reshape(1, -1), bno[l].reshape(1, -1), l < nl - 1, bn)
  return h
